# baseline (speedup 1.0000x reference)
import functools

import jax
import jax.numpy as jnp
from jax.experimental import pallas as pl


def _body(x_ref, wl_ref, bl_ref, wd_ref, bd_ref,
          logits_ref, probs_ref, deltas_ref, *, h, w):
    x = x_ref[...]
    acc = jnp.sum(x, axis=(1, 2)) * (1.0 / (h * w))
    logits = (
        jnp.dot(acc, wl_ref[...], preferred_element_type=jnp.float32)
        + bl_ref[...]
    )
    logits_ref[...] = logits
    m = jnp.max(logits, axis=-1, keepdims=True)
    e = jnp.exp(logits - m)
    probs_ref[...] = e / jnp.sum(e, axis=-1, keepdims=True)
    deltas_ref[...] = (
        jnp.dot(acc, wd_ref[...], preferred_element_type=jnp.float32)
        + bd_ref[...]
    )


def kernel(pooled_rois, W_logits, b_logits, W_delta, b_delta):
    n, h, w, ch = pooled_rois.shape
    ncls = W_logits.shape[1]
    nd = W_delta.shape[1]

    blk = 256
    while n % blk:
        blk //= 2
    grid = (n // blk,)

    bl = b_logits.reshape(1, ncls)
    bd = b_delta.reshape(1, nd)

    body = functools.partial(_body, h=h, w=w)
    logits, probs, deltas = pl.pallas_call(
        body,
        grid=grid,
        in_specs=[
            pl.BlockSpec((blk, h, w, ch), lambda i: (i, 0, 0, 0)),
            pl.BlockSpec((ch, ncls), lambda i: (0, 0)),
            pl.BlockSpec((1, ncls), lambda i: (0, 0)),
            pl.BlockSpec((ch, nd), lambda i: (0, 0)),
            pl.BlockSpec((1, nd), lambda i: (0, 0)),
        ],
        out_specs=[
            pl.BlockSpec((blk, ncls), lambda i: (i, 0)),
            pl.BlockSpec((blk, ncls), lambda i: (i, 0)),
            pl.BlockSpec((blk, nd), lambda i: (i, 0)),
        ],
        out_shape=[
            jax.ShapeDtypeStruct((n, ncls), jnp.float32),
            jax.ShapeDtypeStruct((n, ncls), jnp.float32),
            jax.ShapeDtypeStruct((n, nd), jnp.float32),
        ],
    )(pooled_rois, W_logits, bl, W_delta, bd)
    return (logits, probs, deltas)


# speedup vs baseline: 1.0047x; 1.0047x over previous
import functools

import jax
import jax.numpy as jnp
from jax.experimental import pallas as pl
from jax.experimental.pallas import tpu as pltpu


def _body(x_ref, wl_ref, bl_ref, wd_ref, bd_ref,
          logits_ref, probs_ref, deltas_ref, *, h, w):
    x = x_ref[...]
    acc = jnp.sum(x, axis=(1, 2)) * (1.0 / (h * w))
    logits = (
        jnp.dot(acc, wl_ref[...], preferred_element_type=jnp.float32)
        + bl_ref[...]
    )
    logits_ref[...] = logits
    m = jnp.max(logits, axis=-1, keepdims=True)
    e = jnp.exp(logits - m)
    probs_ref[...] = e / jnp.sum(e, axis=-1, keepdims=True)
    deltas_ref[...] = (
        jnp.dot(acc, wd_ref[...], preferred_element_type=jnp.float32)
        + bd_ref[...]
    )


def kernel(pooled_rois, W_logits, b_logits, W_delta, b_delta):
    n, h, w, ch = pooled_rois.shape
    ncls = W_logits.shape[1]
    nd = W_delta.shape[1]

    blk = 256
    while n % blk:
        blk //= 2
    grid = (n // blk,)

    bl = b_logits.reshape(1, ncls)
    bd = b_delta.reshape(1, nd)

    body = functools.partial(_body, h=h, w=w)
    logits, probs, deltas = pl.pallas_call(
        body,
        grid=grid,
        in_specs=[
            pl.BlockSpec((blk, h, w, ch), lambda i: (i, 0, 0, 0)),
            pl.BlockSpec((ch, ncls), lambda i: (0, 0)),
            pl.BlockSpec((1, ncls), lambda i: (0, 0)),
            pl.BlockSpec((ch, nd), lambda i: (0, 0)),
            pl.BlockSpec((1, nd), lambda i: (0, 0)),
        ],
        out_specs=[
            pl.BlockSpec((blk, ncls), lambda i: (i, 0)),
            pl.BlockSpec((blk, ncls), lambda i: (i, 0)),
            pl.BlockSpec((blk, nd), lambda i: (i, 0)),
        ],
        out_shape=[
            jax.ShapeDtypeStruct((n, ncls), jnp.float32),
            jax.ShapeDtypeStruct((n, ncls), jnp.float32),
            jax.ShapeDtypeStruct((n, nd), jnp.float32),
        ],
        compiler_params=pltpu.CompilerParams(
            dimension_semantics=("parallel",),
        ),
    )(pooled_rois, W_logits, bl, W_delta, bd)
    return (logits, probs, deltas)


# speedup vs baseline: 1.0862x; 1.0811x over previous
import functools

import jax
import jax.numpy as jnp
from jax.experimental import pallas as pl
from jax.experimental.pallas import tpu as pltpu


def _body(x_ref, wl_ref, bl_ref, wd_ref, bd_ref,
          logits_ref, probs_ref, deltas_ref, *, h, w):
    x = x_ref[...]
    blk = x.shape[0] // h
    acc = jnp.sum(x.reshape(blk, h, w, x.shape[2]), axis=(1, 2)) * (1.0 / (h * w))
    logits = (
        jnp.dot(acc, wl_ref[...], preferred_element_type=jnp.float32)
        + bl_ref[...]
    )
    logits_ref[...] = logits
    m = jnp.max(logits, axis=-1, keepdims=True)
    e = jnp.exp(logits - m)
    probs_ref[...] = e / jnp.sum(e, axis=-1, keepdims=True)
    deltas_ref[...] = (
        jnp.dot(acc, wd_ref[...], preferred_element_type=jnp.float32)
        + bd_ref[...]
    )


def kernel(pooled_rois, W_logits, b_logits, W_delta, b_delta):
    n, h, w, ch = pooled_rois.shape
    ncls = W_logits.shape[1]
    nd = W_delta.shape[1]

    blk = 256
    while n % blk:
        blk //= 2
    grid = (n // blk,)

    bl = b_logits.reshape(1, ncls)
    bd = b_delta.reshape(1, nd)
    x3 = pooled_rois.reshape(n * h, w, ch)

    body = functools.partial(_body, h=h, w=w)
    logits, probs, deltas = pl.pallas_call(
        body,
        grid=grid,
        in_specs=[
            pl.BlockSpec((blk * h, w, ch), lambda i: (i, 0, 0)),
            pl.BlockSpec((ch, ncls), lambda i: (0, 0)),
            pl.BlockSpec((1, ncls), lambda i: (0, 0)),
            pl.BlockSpec((ch, nd), lambda i: (0, 0)),
            pl.BlockSpec((1, nd), lambda i: (0, 0)),
        ],
        out_specs=[
            pl.BlockSpec((blk, ncls), lambda i: (i, 0)),
            pl.BlockSpec((blk, ncls), lambda i: (i, 0)),
            pl.BlockSpec((blk, nd), lambda i: (i, 0)),
        ],
        out_shape=[
            jax.ShapeDtypeStruct((n, ncls), jnp.float32),
            jax.ShapeDtypeStruct((n, ncls), jnp.float32),
            jax.ShapeDtypeStruct((n, nd), jnp.float32),
        ],
        compiler_params=pltpu.CompilerParams(
            dimension_semantics=("parallel",),
        ),
    )(x3, W_logits, bl, W_delta, bd)
    return (logits, probs, deltas)


# speedup vs baseline: 3.5475x; 3.2661x over previous
import functools

import jax
import jax.numpy as jnp
from jax.experimental import pallas as pl
from jax.experimental.pallas import tpu as pltpu


def _body(x_ref, wl_ref, bl_ref, wd_ref, bd_ref,
          logits_ref, probs_ref, deltas_ref, *, h, w):
    x = x_ref[...]
    parts = [x[i, j] for i in range(h) for j in range(w)]
    while len(parts) > 1:
        nxt = [a + b for a, b in zip(parts[::2], parts[1::2])]
        if len(parts) % 2:
            nxt.append(parts[-1])
        parts = nxt
    acc = parts[0] * (1.0 / (h * w))
    logits = (
        jnp.dot(acc, wl_ref[...], preferred_element_type=jnp.float32)
        + bl_ref[...]
    )
    logits_ref[...] = logits
    m = jnp.max(logits, axis=-1, keepdims=True)
    e = jnp.exp(logits - m)
    probs_ref[...] = e / jnp.sum(e, axis=-1, keepdims=True)
    deltas_ref[...] = (
        jnp.dot(acc, wd_ref[...], preferred_element_type=jnp.float32)
        + bd_ref[...]
    )


def kernel(pooled_rois, W_logits, b_logits, W_delta, b_delta):
    n, h, w, ch = pooled_rois.shape
    ncls = W_logits.shape[1]
    nd = W_delta.shape[1]

    blk = 256
    while n % blk:
        blk //= 2
    grid = (n // blk,)

    xt = jnp.transpose(pooled_rois, (1, 2, 0, 3))
    bl = b_logits.reshape(1, ncls)
    bd = b_delta.reshape(1, nd)

    body = functools.partial(_body, h=h, w=w)
    logits, probs, deltas = pl.pallas_call(
        body,
        grid=grid,
        in_specs=[
            pl.BlockSpec((h, w, blk, ch), lambda i: (0, 0, i, 0)),
            pl.BlockSpec((ch, ncls), lambda i: (0, 0)),
            pl.BlockSpec((1, ncls), lambda i: (0, 0)),
            pl.BlockSpec((ch, nd), lambda i: (0, 0)),
            pl.BlockSpec((1, nd), lambda i: (0, 0)),
        ],
        out_specs=[
            pl.BlockSpec((blk, ncls), lambda i: (i, 0)),
            pl.BlockSpec((blk, ncls), lambda i: (i, 0)),
            pl.BlockSpec((blk, nd), lambda i: (i, 0)),
        ],
        out_shape=[
            jax.ShapeDtypeStruct((n, ncls), jnp.float32),
            jax.ShapeDtypeStruct((n, ncls), jnp.float32),
            jax.ShapeDtypeStruct((n, nd), jnp.float32),
        ],
        compiler_params=pltpu.CompilerParams(
            dimension_semantics=("parallel",),
        ),
    )(xt, W_logits, bl, W_delta, bd)
    return (logits, probs, deltas)


# speedup vs baseline: 4.0830x; 1.1509x over previous
import functools

import jax
import jax.numpy as jnp
from jax.experimental import pallas as pl
from jax.experimental.pallas import tpu as pltpu


def _body(x_ref, wlt_ref, bl_ref, wdt_ref, bd_ref,
          logits_ref, probs_ref, deltas_ref, *, h, w):
    x = x_ref[...]
    parts = [x[i, j] for i in range(h) for j in range(w)]
    while len(parts) > 1:
        nxt = [a + b for a, b in zip(parts[::2], parts[1::2])]
        if len(parts) % 2:
            nxt.append(parts[-1])
        parts = nxt
    acc_t = jnp.transpose(parts[0] * (1.0 / (h * w)))
    logits_t = (
        jnp.dot(wlt_ref[...], acc_t, preferred_element_type=jnp.float32)
        + jnp.transpose(bl_ref[...])
    )
    logits_ref[...] = logits_t
    m = jnp.max(logits_t, axis=0, keepdims=True)
    e = jnp.exp(logits_t - m)
    probs_ref[...] = e / jnp.sum(e, axis=0, keepdims=True)
    deltas_ref[...] = (
        jnp.dot(wdt_ref[...], acc_t, preferred_element_type=jnp.float32)
        + jnp.transpose(bd_ref[...])
    )


def kernel(pooled_rois, W_logits, b_logits, W_delta, b_delta):
    n, h, w, ch = pooled_rois.shape
    ncls = W_logits.shape[1]
    nd = W_delta.shape[1]

    blk = 256
    while n % blk:
        blk //= 2
    grid = (n // blk,)

    xt = jnp.transpose(pooled_rois, (1, 2, 0, 3))
    wlt = jnp.transpose(W_logits)
    wdt = jnp.transpose(W_delta)
    bl = b_logits.reshape(1, ncls)
    bd = b_delta.reshape(1, nd)

    body = functools.partial(_body, h=h, w=w)
    logits_t, probs_t, deltas_t = pl.pallas_call(
        body,
        grid=grid,
        in_specs=[
            pl.BlockSpec((h, w, blk, ch), lambda i: (0, 0, i, 0)),
            pl.BlockSpec((ncls, ch), lambda i: (0, 0)),
            pl.BlockSpec((1, ncls), lambda i: (0, 0)),
            pl.BlockSpec((nd, ch), lambda i: (0, 0)),
            pl.BlockSpec((1, nd), lambda i: (0, 0)),
        ],
        out_specs=[
            pl.BlockSpec((ncls, blk), lambda i: (0, i)),
            pl.BlockSpec((ncls, blk), lambda i: (0, i)),
            pl.BlockSpec((nd, blk), lambda i: (0, i)),
        ],
        out_shape=[
            jax.ShapeDtypeStruct((ncls, n), jnp.float32),
            jax.ShapeDtypeStruct((ncls, n), jnp.float32),
            jax.ShapeDtypeStruct((nd, n), jnp.float32),
        ],
        compiler_params=pltpu.CompilerParams(
            dimension_semantics=("parallel",),
        ),
    )(xt, wlt, bl, wdt, bd)
    return (
        jnp.transpose(logits_t),
        jnp.transpose(probs_t),
        jnp.transpose(deltas_t),
    )
